# Initial kernel scaffold; baseline (speedup 1.0000x reference)
import jax
import jax.numpy as jnp
from jax.experimental import pallas as pl


def kernel(x, edge_index, W1, b1, W2, b2, Wfc, bfc):
    raise NotImplementedError("write your pallas kernel here")


# speedup vs baseline: 5.8566x; 5.8566x over previous
import functools

import jax
import jax.numpy as jnp
from jax import lax
from jax.experimental import pallas as pl
from jax.experimental.pallas import tpu as pltpu
from jax.experimental.pallas import tpu_sc as plsc

N = 10000
NPAD = 10240
E = 160000
EPAD = 163840
EXTRA = EPAD - E
PAD_ROW = 10016
EB = EPAD // 16 // 128
RPT = NPAD // 16
NB = 512
GRID = NPAD // NB
D_HID = 512
NCHUNK = 4

_mesh = plsc.VectorSubcoreMesh(core_axis_name="c", subcore_axis_name="s")


@functools.partial(
    pl.kernel,
    mesh=_mesh,
    out_type=jax.ShapeDtypeStruct((NPAD,), jnp.float32),
    scratch_types=[
        pltpu.VMEM((EB, 128), jnp.int32),
        pltpu.VMEM((RPT,), jnp.float32),
        pltpu.VMEM((128,), jnp.float32),
        pltpu.VMEM_SHARED((NPAD,), jnp.float32),
    ],
)
def _sc_deg(dstr_hbm, deg_hbm, dst_v, init_v, ones_v, deg_acc):
    core = lax.axis_index("c")
    sid = lax.axis_index("s")

    @pl.when(core == 0)
    def _():
        def fill(i, c):
            init_v[pl.ds(i * 16, 16)] = jnp.full((16,), 1.0, jnp.float32)
            return c
        lax.fori_loop(0, RPT // 16, fill, 0)
        for k in range(8):
            ones_v[pl.ds(k * 16, 16)] = jnp.full((16,), 1.0, jnp.float32)
        pltpu.sync_copy(dstr_hbm.at[sid], dst_v)
        pltpu.sync_copy(init_v, deg_acc.at[pl.ds(sid * RPT, RPT)])
        plsc.subcore_barrier()

        def body(j, c):
            pltpu.sync_copy(ones_v, deg_acc.at[dst_v.at[j]], add=True)
            return c
        lax.fori_loop(0, EB, body, 0)
        plsc.subcore_barrier()
        pltpu.sync_copy(deg_acc.at[pl.ds(sid * RPT, RPT)],
                        deg_hbm.at[pl.ds(sid * RPT, RPT)])


@functools.partial(
    pl.kernel,
    mesh=_mesh,
    out_type=jax.ShapeDtypeStruct((NCHUNK, NPAD, 128), jnp.float32),
    scratch_types=[
        pltpu.VMEM((EB, 128), jnp.int32),
        pltpu.VMEM((EB, 128), jnp.int32),
        pltpu.VMEM((128, 128), jnp.float32),
        pltpu.VMEM_SHARED((NPAD, 128), jnp.float32),
        pltpu.SemaphoreType.DMA,
    ],
)
def _sc_agg(p_hbm, srcr_hbm, dstr_hbm, out_hbm, src_v, dst_v, gbuf, accum, sem):
    core = lax.axis_index("c")
    sid = lax.axis_index("s")
    pltpu.sync_copy(srcr_hbm.at[sid], src_v)
    pltpu.sync_copy(dstr_hbm.at[sid], dst_v)
    for ci in range(2):
        chunk = core * 2 + ci
        pltpu.sync_copy(p_hbm.at[chunk, pl.ds(sid * RPT, RPT)],
                        accum.at[pl.ds(sid * RPT, RPT)])
        plsc.subcore_barrier()

        def body(j, c):
            pltpu.async_copy(p_hbm.at[chunk].at[src_v.at[j]], gbuf, sem).wait()
            pltpu.sync_copy(gbuf, accum.at[dst_v.at[j]], add=True)
            return c
        lax.fori_loop(0, EB, body, 0)
        plsc.subcore_barrier()
        pltpu.sync_copy(accum.at[pl.ds(sid * RPT, RPT)],
                        out_hbm.at[chunk, pl.ds(sid * RPT, RPT)])


def _tc1_body(x_ref, w_ref, deg_ref, out_ref):
    h = jnp.dot(x_ref[...], w_ref[...], preferred_element_type=jnp.float32)
    dinv = lax.rsqrt(deg_ref[...])
    p = h * dinv
    for c in range(NCHUNK):
        out_ref[c] = p[:, c * 128:(c + 1) * 128]


def _tc1(xp, W1, deg2):
    return pl.pallas_call(
        _tc1_body,
        grid=(GRID,),
        in_specs=[
            pl.BlockSpec((NB, 256), lambda i: (i, 0)),
            pl.BlockSpec((256, D_HID), lambda i: (0, 0)),
            pl.BlockSpec((NB, 1), lambda i: (i, 0)),
        ],
        out_specs=pl.BlockSpec((NCHUNK, NB, 128), lambda i: (0, i, 0)),
        out_shape=jax.ShapeDtypeStruct((NCHUNK, NPAD, 128), jnp.float32),
    )(xp, W1, deg2)


def _tc2_body(s_ref, deg_ref, b_ref, w_ref, out_ref):
    dinv = lax.rsqrt(deg_ref[...])
    acc = jnp.zeros((NB, D_HID), jnp.float32)
    for c in range(NCHUNK):
        h_c = jnp.maximum(s_ref[c] * dinv + b_ref[:, c * 128:(c + 1) * 128], 0.0)
        acc = acc + jnp.dot(h_c, w_ref[pl.ds(c * 128, 128), :],
                            preferred_element_type=jnp.float32)
    p = acc * dinv
    for c in range(NCHUNK):
        out_ref[c] = p[:, c * 128:(c + 1) * 128]


def _tc2(s1, deg2, b1, W2):
    return pl.pallas_call(
        _tc2_body,
        grid=(GRID,),
        in_specs=[
            pl.BlockSpec((NCHUNK, NB, 128), lambda i: (0, i, 0)),
            pl.BlockSpec((NB, 1), lambda i: (i, 0)),
            pl.BlockSpec((1, D_HID), lambda i: (0, 0)),
            pl.BlockSpec((D_HID, D_HID), lambda i: (0, 0)),
        ],
        out_specs=pl.BlockSpec((NCHUNK, NB, 128), lambda i: (0, i, 0)),
        out_shape=jax.ShapeDtypeStruct((NCHUNK, NPAD, 128), jnp.float32),
    )(s1, deg2, b1, W2)


def _tc3_body(s_ref, deg_ref, b_ref, wfc_ref, bfc_ref, out_ref, acc_ref):
    i = pl.program_id(0)
    dinv = lax.rsqrt(deg_ref[...])
    rows = lax.broadcasted_iota(jnp.int32, (NB, 1), 0) + i * NB
    mask = rows < N
    parts = []
    for c in range(NCHUNK):
        h_c = jnp.maximum(s_ref[c] * dinv + b_ref[:, c * 128:(c + 1) * 128], 0.0)
        h_c = jnp.where(mask, h_c, 0.0)
        parts.append(jnp.sum(h_c, axis=0, keepdims=True))
    part = jnp.concatenate(parts, axis=1)

    @pl.when(i == 0)
    def _():
        acc_ref[...] = part

    @pl.when(i > 0)
    def _():
        acc_ref[...] = acc_ref[...] + part

    @pl.when(i == pl.num_programs(0) - 1)
    def _():
        pooled = acc_ref[...] * (1.0 / N)
        out_ref[...] = jnp.dot(pooled, wfc_ref[...],
                               preferred_element_type=jnp.float32) + bfc_ref[...]


def _tc3(s2, deg2, b2, Wfc, bfc):
    return pl.pallas_call(
        _tc3_body,
        grid=(GRID,),
        in_specs=[
            pl.BlockSpec((NCHUNK, NB, 128), lambda i: (0, i, 0)),
            pl.BlockSpec((NB, 1), lambda i: (i, 0)),
            pl.BlockSpec((1, D_HID), lambda i: (0, 0)),
            pl.BlockSpec((D_HID, 256), lambda i: (0, 0)),
            pl.BlockSpec((1, 256), lambda i: (0, 0)),
        ],
        out_specs=pl.BlockSpec((1, 256), lambda i: (0, 0)),
        out_shape=jax.ShapeDtypeStruct((1, 256), jnp.float32),
        scratch_shapes=[pltpu.VMEM((1, D_HID), jnp.float32)],
    )(s2, deg2, b2, Wfc, bfc)


def kernel(x, edge_index, W1, b1, W2, b2, Wfc, bfc):
    ei = edge_index.astype(jnp.int32)
    src = jnp.concatenate([ei[0], jnp.zeros((EXTRA,), jnp.int32)])
    dst = jnp.concatenate([ei[1], jnp.full((EXTRA,), PAD_ROW, jnp.int32)])
    src_r = src.reshape(16, EB, 128)
    dst_r = dst.reshape(16, EB, 128)
    xp = jnp.pad(x, ((0, NPAD - N), (0, 0)))

    deg = _sc_deg(dst_r)
    deg2 = deg.reshape(NPAD, 1)
    p1 = _tc1(xp, W1, deg2)
    s1 = _sc_agg(p1, src_r, dst_r)
    p2 = _tc2(s1, deg2, b1.reshape(1, D_HID), W2)
    s2 = _sc_agg(p2, src_r, dst_r)
    out = _tc3(s2, deg2, b2.reshape(1, D_HID), Wfc, bfc.reshape(1, 256))
    return out.reshape(256)


# speedup vs baseline: 5.8658x; 1.0016x over previous
import functools

import jax
import jax.numpy as jnp
from jax import lax
from jax.experimental import pallas as pl
from jax.experimental.pallas import tpu as pltpu
from jax.experimental.pallas import tpu_sc as plsc

N = 10000
NPAD = 10240
E = 160000
EPAD = 163840
EXTRA = EPAD - E
PAD_ROW = 10016
BATCH = 128
EB = EPAD // 16 // BATCH
RPT = NPAD // 16
NB = 512
GRID = NPAD // NB
D_HID = 512
NCHUNK = 4

_mesh = plsc.VectorSubcoreMesh(core_axis_name="c", subcore_axis_name="s")


@functools.partial(
    pl.kernel,
    mesh=_mesh,
    out_type=jax.ShapeDtypeStruct((NPAD,), jnp.float32),
    scratch_types=[
        pltpu.VMEM((EB, BATCH), jnp.int32),
        pltpu.VMEM((RPT,), jnp.float32),
        pltpu.VMEM((BATCH,), jnp.float32),
        pltpu.VMEM_SHARED((NPAD,), jnp.float32),
    ],
)
def _sc_deg(dstr_hbm, deg_hbm, dst_v, init_v, ones_v, deg_acc):
    core = lax.axis_index("c")
    sid = lax.axis_index("s")

    @pl.when(core == 0)
    def _():
        def fill(i, c):
            init_v[pl.ds(i * 16, 16)] = jnp.full((16,), 1.0, jnp.float32)
            return c
        lax.fori_loop(0, RPT // 16, fill, 0)
        for k in range(BATCH // 16):
            ones_v[pl.ds(k * 16, 16)] = jnp.full((16,), 1.0, jnp.float32)
        pltpu.sync_copy(dstr_hbm.at[sid], dst_v)
        pltpu.sync_copy(init_v, deg_acc.at[pl.ds(sid * RPT, RPT)])
        plsc.subcore_barrier()

        def body(j, c):
            pltpu.sync_copy(ones_v, deg_acc.at[dst_v.at[j]], add=True)
            return c
        lax.fori_loop(0, EB, body, 0)
        plsc.subcore_barrier()
        pltpu.sync_copy(deg_acc.at[pl.ds(sid * RPT, RPT)],
                        deg_hbm.at[pl.ds(sid * RPT, RPT)])


@functools.partial(
    pl.kernel,
    mesh=_mesh,
    out_type=jax.ShapeDtypeStruct((NCHUNK, NPAD, 128), jnp.float32),
    scratch_types=[
        pltpu.VMEM((EB, BATCH), jnp.int32),
        pltpu.VMEM((EB, BATCH), jnp.int32),
        pltpu.VMEM((BATCH, 128), jnp.float32),
        pltpu.VMEM((BATCH, 128), jnp.float32),
        pltpu.VMEM_SHARED((NPAD, 128), jnp.float32),
        pltpu.SemaphoreType.DMA,
        pltpu.SemaphoreType.DMA,
    ],
)
def _sc_agg(p_hbm, srcr_hbm, dstr_hbm, out_hbm, src_v, dst_v, gbuf0, gbuf1,
            accum, sem0, sem1):
    core = lax.axis_index("c")
    sid = lax.axis_index("s")
    pltpu.sync_copy(srcr_hbm.at[sid], src_v)
    pltpu.sync_copy(dstr_hbm.at[sid], dst_v)
    gbufs = (gbuf0, gbuf1)
    sems = (sem0, sem1)
    UNROLL = 10
    for ci in range(2):
        chunk = core * 2 + ci
        tbl = p_hbm.at[chunk]
        pltpu.sync_copy(p_hbm.at[chunk, pl.ds(sid * RPT, RPT)],
                        accum.at[pl.ds(sid * RPT, RPT)])
        plsc.subcore_barrier()

        def body(k, c):
            j0 = k * UNROLL
            for t in range(UNROLL):
                pltpu.async_copy(tbl.at[src_v.at[j0 + t]], gbufs[0],
                                 sems[0]).wait()
                pltpu.sync_copy(gbufs[0], accum.at[dst_v.at[j0 + t]],
                                add=True)
            return c
        lax.fori_loop(0, EB // UNROLL, body, 0)
        plsc.subcore_barrier()
        pltpu.sync_copy(accum.at[pl.ds(sid * RPT, RPT)],
                        out_hbm.at[chunk, pl.ds(sid * RPT, RPT)])


def _tc1_body(x_ref, w_ref, deg_ref, out_ref):
    h = jnp.dot(x_ref[...], w_ref[...], preferred_element_type=jnp.float32)
    dinv = lax.rsqrt(deg_ref[...])
    p = h * dinv
    for c in range(NCHUNK):
        out_ref[c] = p[:, c * 128:(c + 1) * 128]


def _tc1(xp, W1, deg2):
    return pl.pallas_call(
        _tc1_body,
        grid=(GRID,),
        in_specs=[
            pl.BlockSpec((NB, 256), lambda i: (i, 0)),
            pl.BlockSpec((256, D_HID), lambda i: (0, 0)),
            pl.BlockSpec((NB, 1), lambda i: (i, 0)),
        ],
        out_specs=pl.BlockSpec((NCHUNK, NB, 128), lambda i: (0, i, 0)),
        out_shape=jax.ShapeDtypeStruct((NCHUNK, NPAD, 128), jnp.float32),
    )(xp, W1, deg2)


def _tc2_body(s_ref, deg_ref, b_ref, w_ref, out_ref):
    dinv = lax.rsqrt(deg_ref[...])
    acc = jnp.zeros((NB, D_HID), jnp.float32)
    for c in range(NCHUNK):
        h_c = jnp.maximum(s_ref[c] * dinv + b_ref[:, c * 128:(c + 1) * 128], 0.0)
        acc = acc + jnp.dot(h_c, w_ref[pl.ds(c * 128, 128), :],
                            preferred_element_type=jnp.float32)
    p = acc * dinv
    for c in range(NCHUNK):
        out_ref[c] = p[:, c * 128:(c + 1) * 128]


def _tc2(s1, deg2, b1, W2):
    return pl.pallas_call(
        _tc2_body,
        grid=(GRID,),
        in_specs=[
            pl.BlockSpec((NCHUNK, NB, 128), lambda i: (0, i, 0)),
            pl.BlockSpec((NB, 1), lambda i: (i, 0)),
            pl.BlockSpec((1, D_HID), lambda i: (0, 0)),
            pl.BlockSpec((D_HID, D_HID), lambda i: (0, 0)),
        ],
        out_specs=pl.BlockSpec((NCHUNK, NB, 128), lambda i: (0, i, 0)),
        out_shape=jax.ShapeDtypeStruct((NCHUNK, NPAD, 128), jnp.float32),
    )(s1, deg2, b1, W2)


def _tc3_body(s_ref, deg_ref, b_ref, wfc_ref, bfc_ref, out_ref, acc_ref):
    i = pl.program_id(0)
    dinv = lax.rsqrt(deg_ref[...])
    rows = lax.broadcasted_iota(jnp.int32, (NB, 1), 0) + i * NB
    mask = rows < N
    parts = []
    for c in range(NCHUNK):
        h_c = jnp.maximum(s_ref[c] * dinv + b_ref[:, c * 128:(c + 1) * 128], 0.0)
        h_c = jnp.where(mask, h_c, 0.0)
        parts.append(jnp.sum(h_c, axis=0, keepdims=True))
    part = jnp.concatenate(parts, axis=1)

    @pl.when(i == 0)
    def _():
        acc_ref[...] = part

    @pl.when(i > 0)
    def _():
        acc_ref[...] = acc_ref[...] + part

    @pl.when(i == pl.num_programs(0) - 1)
    def _():
        pooled = acc_ref[...] * (1.0 / N)
        out_ref[...] = jnp.dot(pooled, wfc_ref[...],
                               preferred_element_type=jnp.float32) + bfc_ref[...]


def _tc3(s2, deg2, b2, Wfc, bfc):
    return pl.pallas_call(
        _tc3_body,
        grid=(GRID,),
        in_specs=[
            pl.BlockSpec((NCHUNK, NB, 128), lambda i: (0, i, 0)),
            pl.BlockSpec((NB, 1), lambda i: (i, 0)),
            pl.BlockSpec((1, D_HID), lambda i: (0, 0)),
            pl.BlockSpec((D_HID, 256), lambda i: (0, 0)),
            pl.BlockSpec((1, 256), lambda i: (0, 0)),
        ],
        out_specs=pl.BlockSpec((1, 256), lambda i: (0, 0)),
        out_shape=jax.ShapeDtypeStruct((1, 256), jnp.float32),
        scratch_shapes=[pltpu.VMEM((1, D_HID), jnp.float32)],
    )(s2, deg2, b2, Wfc, bfc)


def kernel(x, edge_index, W1, b1, W2, b2, Wfc, bfc):
    ei = edge_index.astype(jnp.int32)
    src = jnp.concatenate([ei[0], jnp.zeros((EXTRA,), jnp.int32)])
    dst = jnp.concatenate([ei[1], jnp.full((EXTRA,), PAD_ROW, jnp.int32)])
    src_r = src.reshape(16, EB, BATCH)
    dst_r = dst.reshape(16, EB, BATCH)
    xp = jnp.pad(x, ((0, NPAD - N), (0, 0)))

    deg = _sc_deg(dst_r)
    deg2 = deg.reshape(NPAD, 1)
    p1 = _tc1(xp, W1, deg2)
    s1 = _sc_agg(p1, src_r, dst_r)
    p2 = _tc2(s1, deg2, b1.reshape(1, D_HID), W2)
    s2 = _sc_agg(p2, src_r, dst_r)
    out = _tc3(s2, deg2, b2.reshape(1, D_HID), Wfc, bfc.reshape(1, 256))
    return out.reshape(256)


# speedup vs baseline: 6.9839x; 1.1906x over previous
import functools

import jax
import jax.numpy as jnp
from jax import lax
from jax.experimental import pallas as pl
from jax.experimental.pallas import tpu as pltpu
from jax.experimental.pallas import tpu_sc as plsc

N = 10000
NPAD = 10240
E = 160000
EPAD = 163840
EXTRA = EPAD - E
PAD_ROW = 10016
BATCH = 128
EB = EPAD // 16 // BATCH
SEC = 16
RPT = NPAD // 16
NB = 512
GRID = NPAD // NB
D_HID = 512
NCHUNK = 4

_mesh = plsc.VectorSubcoreMesh(core_axis_name="c", subcore_axis_name="s")


@functools.partial(
    pl.kernel,
    mesh=_mesh,
    out_type=jax.ShapeDtypeStruct((NPAD,), jnp.float32),
    scratch_types=[
        pltpu.VMEM((EB, BATCH), jnp.int32),
        pltpu.VMEM((RPT,), jnp.float32),
        pltpu.VMEM((BATCH,), jnp.float32),
        pltpu.VMEM_SHARED((NPAD,), jnp.float32),
    ],
)
def _sc_deg(dstr_hbm, deg_hbm, dst_v, init_v, ones_v, deg_acc):
    core = lax.axis_index("c")
    sid = lax.axis_index("s")

    @pl.when(core == 0)
    def _():
        def fill(i, c):
            init_v[pl.ds(i * 16, 16)] = jnp.full((16,), 1.0, jnp.float32)
            return c
        lax.fori_loop(0, RPT // 16, fill, 0)
        for k in range(BATCH // 16):
            ones_v[pl.ds(k * 16, 16)] = jnp.full((16,), 1.0, jnp.float32)
        pltpu.sync_copy(dstr_hbm.at[sid], dst_v)
        pltpu.sync_copy(init_v, deg_acc.at[pl.ds(sid * RPT, RPT)])
        plsc.subcore_barrier()

        def body(j, c):
            pltpu.sync_copy(ones_v, deg_acc.at[dst_v.at[j]], add=True)
            return c
        lax.fori_loop(0, EB, body, 0)
        plsc.subcore_barrier()
        pltpu.sync_copy(deg_acc.at[pl.ds(sid * RPT, RPT)],
                        deg_hbm.at[pl.ds(sid * RPT, RPT)])


@functools.partial(
    pl.kernel,
    mesh=_mesh,
    out_type=jax.ShapeDtypeStruct((NCHUNK, NPAD, 128), jnp.float32),
    scratch_types=[
        pltpu.VMEM((SEC, BATCH), jnp.int32),
        pltpu.VMEM((SEC, BATCH), jnp.int32),
        pltpu.VMEM((2, BATCH, 128), jnp.float32),
        pltpu.VMEM_SHARED((NPAD, 128), jnp.float32),
        pltpu.SemaphoreType.DMA,
    ],
)
def _sc_agg(p_hbm, srcr_hbm, dstr_hbm, out_hbm, src_v, dst_v, gbuf, accum,
            sem):
    core = lax.axis_index("c")
    sid = lax.axis_index("s")
    for ci in range(2):
        chunk = core * 2 + ci
        tbl = p_hbm.at[chunk]
        pltpu.sync_copy(p_hbm.at[chunk, pl.ds(sid * RPT, RPT)],
                        accum.at[pl.ds(sid * RPT, RPT)])
        plsc.subcore_barrier()

        def sec_body(sc, c):
            pltpu.sync_copy(srcr_hbm.at[sid, pl.ds(sc * SEC, SEC)], src_v)
            pltpu.sync_copy(dstr_hbm.at[sid, pl.ds(sc * SEC, SEC)], dst_v)
            pltpu.async_copy(tbl.at[src_v.at[0]], gbuf.at[0], sem)

            def body(j, c2):
                par = lax.rem(j, 2)
                nxt = lax.rem(j + 1, 2)

                @pl.when(j + 1 < SEC)
                def _():
                    pltpu.async_copy(tbl.at[src_v.at[j + 1]], gbuf.at[nxt],
                                     sem)
                pltpu.make_async_copy(tbl.at[src_v.at[j]], gbuf.at[par],
                                      sem).wait()
                pltpu.sync_copy(gbuf.at[par], accum.at[dst_v.at[j]], add=True)
                return c2
            lax.fori_loop(0, SEC, body, 0)
            return c
        lax.fori_loop(0, EB // SEC, sec_body, 0)
        plsc.subcore_barrier()
        pltpu.sync_copy(accum.at[pl.ds(sid * RPT, RPT)],
                        out_hbm.at[chunk, pl.ds(sid * RPT, RPT)])


def _tc1_body(x_ref, w_ref, deg_ref, out_ref):
    h = jnp.dot(x_ref[...], w_ref[...], preferred_element_type=jnp.float32)
    dinv = lax.rsqrt(deg_ref[...])
    p = h * dinv
    for c in range(NCHUNK):
        out_ref[c] = p[:, c * 128:(c + 1) * 128]


def _tc1(xp, W1, deg2):
    return pl.pallas_call(
        _tc1_body,
        grid=(GRID,),
        in_specs=[
            pl.BlockSpec((NB, 256), lambda i: (i, 0)),
            pl.BlockSpec((256, D_HID), lambda i: (0, 0)),
            pl.BlockSpec((NB, 1), lambda i: (i, 0)),
        ],
        out_specs=pl.BlockSpec((NCHUNK, NB, 128), lambda i: (0, i, 0)),
        out_shape=jax.ShapeDtypeStruct((NCHUNK, NPAD, 128), jnp.float32),
    )(xp, W1, deg2)


def _tc2_body(s_ref, deg_ref, b_ref, w_ref, out_ref):
    dinv = lax.rsqrt(deg_ref[...])
    s = jnp.concatenate([s_ref[c] for c in range(NCHUNK)], axis=1)
    h1 = jnp.maximum(s * dinv + b_ref[...], 0.0)
    p = jnp.dot(h1, w_ref[...], preferred_element_type=jnp.float32) * dinv
    for c in range(NCHUNK):
        out_ref[c] = p[:, c * 128:(c + 1) * 128]


def _tc2(s1, deg2, b1, W2):
    return pl.pallas_call(
        _tc2_body,
        grid=(GRID,),
        in_specs=[
            pl.BlockSpec((NCHUNK, NB, 128), lambda i: (0, i, 0)),
            pl.BlockSpec((NB, 1), lambda i: (i, 0)),
            pl.BlockSpec((1, D_HID), lambda i: (0, 0)),
            pl.BlockSpec((D_HID, D_HID), lambda i: (0, 0)),
        ],
        out_specs=pl.BlockSpec((NCHUNK, NB, 128), lambda i: (0, i, 0)),
        out_shape=jax.ShapeDtypeStruct((NCHUNK, NPAD, 128), jnp.float32),
    )(s1, deg2, b1, W2)


def _tc3_body(s_ref, deg_ref, b_ref, wfc_ref, bfc_ref, out_ref, acc_ref):
    i = pl.program_id(0)
    dinv = lax.rsqrt(deg_ref[...])
    rows = lax.broadcasted_iota(jnp.int32, (NB, 1), 0) + i * NB
    mask = rows < N
    s = jnp.concatenate([s_ref[c] for c in range(NCHUNK)], axis=1)
    h2 = jnp.maximum(s * dinv + b_ref[...], 0.0)
    h2 = jnp.where(mask, h2, 0.0)
    part = jnp.sum(h2, axis=0, keepdims=True)

    @pl.when(i == 0)
    def _():
        acc_ref[...] = part

    @pl.when(i > 0)
    def _():
        acc_ref[...] = acc_ref[...] + part

    @pl.when(i == pl.num_programs(0) - 1)
    def _():
        pooled = acc_ref[...] * (1.0 / N)
        out_ref[...] = jnp.dot(pooled, wfc_ref[...],
                               preferred_element_type=jnp.float32) + bfc_ref[...]


def _tc3(s2, deg2, b2, Wfc, bfc):
    return pl.pallas_call(
        _tc3_body,
        grid=(GRID,),
        in_specs=[
            pl.BlockSpec((NCHUNK, NB, 128), lambda i: (0, i, 0)),
            pl.BlockSpec((NB, 1), lambda i: (i, 0)),
            pl.BlockSpec((1, D_HID), lambda i: (0, 0)),
            pl.BlockSpec((D_HID, 256), lambda i: (0, 0)),
            pl.BlockSpec((1, 256), lambda i: (0, 0)),
        ],
        out_specs=pl.BlockSpec((1, 256), lambda i: (0, 0)),
        out_shape=jax.ShapeDtypeStruct((1, 256), jnp.float32),
        scratch_shapes=[pltpu.VMEM((1, D_HID), jnp.float32)],
    )(s2, deg2, b2, Wfc, bfc)


def kernel(x, edge_index, W1, b1, W2, b2, Wfc, bfc):
    ei = edge_index.astype(jnp.int32)
    src = jnp.concatenate([ei[0], jnp.zeros((EXTRA,), jnp.int32)])
    dst = jnp.concatenate([ei[1], jnp.full((EXTRA,), PAD_ROW, jnp.int32)])
    src_r = src.reshape(16, EB, BATCH)
    dst_r = dst.reshape(16, EB, BATCH)
    xp = jnp.pad(x, ((0, NPAD - N), (0, 0)))

    deg = _sc_deg(dst_r)
    deg2 = deg.reshape(NPAD, 1)
    p1 = _tc1(xp, W1, deg2)
    s1 = _sc_agg(p1, src_r, dst_r)
    p2 = _tc2(s1, deg2, b1.reshape(1, D_HID), W2)
    s2 = _sc_agg(p2, src_r, dst_r)
    out = _tc3(s2, deg2, b2.reshape(1, D_HID), Wfc, bfc.reshape(1, 256))
    return out.reshape(256)
